# baseline (speedup 1.0000x reference)
import functools

import jax
import jax.numpy as jnp
from jax import lax
from jax.experimental import pallas as pl
from jax.experimental.pallas import tpu as pltpu
from jax.experimental.pallas import tpu_sc as plsc

_BUCKETS = 128
_EMB = 32
_NW = 32
_CHUNK = 128
_LANES = 16


def _round_half_even_clip(y):
    k = y.astype(jnp.int32)
    r = y - k.astype(jnp.float32)
    half = jnp.float32(0.5)
    up = (r > half) | ((r == half) & ((k & 1) == 1))
    t = k + jnp.where(up, 1, 0)
    return jnp.minimum(jnp.maximum(t, 0), _BUCKETS - 1)


_TPAD = 56


def _sc_embed(flat_coords, coord_embed, batch, seq):
    b_per_w = batch // _NW
    b_per_c = 2
    rows_c = b_per_c * _TPAD
    n_chunks = b_per_w // b_per_c
    n_pairs = n_chunks // 2

    mesh = plsc.VectorSubcoreMesh(core_axis_name="c", subcore_axis_name="s")

    @functools.partial(
        pl.kernel,
        mesh=mesh,
        out_type=jax.ShapeDtypeStruct((batch * _TPAD, 4 * _EMB), jnp.float32),
        scratch_types=[
            pltpu.VMEM((b_per_w * seq * 4,), jnp.float32),
            pltpu.VMEM((_BUCKETS, _EMB), jnp.float32),
            pltpu.VMEM((7 * 4 * _LANES,), jnp.int32),
            pltpu.VMEM((rows_c, 4 * _EMB), jnp.float32),
            pltpu.VMEM((rows_c, 4 * _EMB), jnp.float32),
            pltpu.SemaphoreType.DMA,
        ],
        compiler_params=pltpu.CompilerParams(
            use_tc_tiling_on_sc=False, needs_layout_passes=False),
    )
    def k(coords_hbm, table_hbm, out_hbm, coords_v, table_v, idsb, ob0, ob1,
          sem):
        wid = lax.axis_index("s") * 2 + lax.axis_index("c")
        obufs = [ob0, ob1]
        iota = lax.iota(jnp.int32, _LANES)
        pltpu.sync_copy(table_hbm, table_v)
        pltpu.sync_copy(
            coords_hbm.at[pl.ds(wid * (b_per_w * seq * 4), b_per_w * seq * 4)],
            coords_v)

        def fill(obuf, j):
            cbase = j * (b_per_c * seq * 4)

            def ids_body(g, carry):
                rvec = iota + g * _LANES
                b_l = jnp.where(rvec >= _TPAD, 1, 0)
                t = jnp.minimum(rvec - b_l * _TPAD, seq - 1)
                cidx = cbase + b_l * (seq * 4) + t * 4
                for c in range(4):
                    xi = plsc.load_gather(coords_v, [cidx + c])
                    idsb[pl.ds(g * 4 * _LANES + c * _LANES, _LANES)] = (
                        _round_half_even_clip(xi * jnp.float32(_BUCKETS - 1)))
                return carry

            lax.fori_loop(0, rows_c // _LANES, ids_body, 0)

            def lk_body(g, carry):
                rows0 = g * _LANES
                base = g * 4 * _LANES
                for tl in range(_LANES):
                    for c in range(4):
                        spl = plsc.load_gather(
                            idsb, [jnp.full((_LANES,), 1, jnp.int32)
                                   * (base + c * _LANES + tl)])
                        for h in range(2):
                            v = plsc.load_gather(
                                table_v, [spl, iota + h * _LANES])
                            obuf[rows0 + tl,
                                 pl.ds(c * _EMB + h * _LANES, _LANES)] = v
                return carry

            lax.fori_loop(0, rows_c // _LANES, lk_body, 0)

        def pair_body(jj, carry):
            for b in range(2):
                j = jj * 2 + b

                @pl.when(jj > 0)
                def _wait():
                    pltpu.make_async_copy(
                        out_hbm.at[pl.ds(0, rows_c), :], obufs[b], sem).wait()

                fill(obufs[b], j)
                pltpu.async_copy(
                    obufs[b],
                    out_hbm.at[pl.ds(wid * (b_per_w * _TPAD) + j * rows_c,
                                     rows_c), :],
                    sem)
            return carry

        lax.fori_loop(0, n_pairs, pair_body, 0)
        for b in range(2):
            pltpu.make_async_copy(
                out_hbm.at[pl.ds(0, rows_c), :], obufs[b], sem).wait()

    return k(flat_coords, coord_embed)


def _tc_mlp(embs3, w1, b1, w2, b2, gamma, beta, b_tile):
    batch, tpad, d_in = embs3.shape
    seq = 50
    d_hid = w1.shape[1]
    d_out = w2.shape[1]

    def body(e_ref, w1_ref, b1_ref, w2_ref, b2_ref, g_ref, be_ref, o_ref):
        e = e_ref[...][:, :seq, :].reshape(b_tile * seq, d_in)
        h = jnp.dot(e, w1_ref[...],
                    preferred_element_type=jnp.float32) + b1_ref[...]
        h = h * 0.5 * (1.0 + lax.erf(h * jnp.float32(0.7071067811865476)))
        y = jnp.dot(h, w2_ref[...],
                    preferred_element_type=jnp.float32) + b2_ref[...]
        mu = jnp.mean(y, axis=-1, keepdims=True)
        var = jnp.mean((y - mu) * (y - mu), axis=-1, keepdims=True)
        y = (y - mu) / jnp.sqrt(var + 1e-5) * g_ref[...] + be_ref[...]
        o_ref[...] = y.reshape(b_tile, seq, d_out)

    return pl.pallas_call(
        body,
        grid=(batch // b_tile,),
        in_specs=[
            pl.BlockSpec((b_tile, tpad, d_in), lambda i: (i, 0, 0)),
            pl.BlockSpec((d_in, d_hid), lambda i: (0, 0)),
            pl.BlockSpec((1, d_hid), lambda i: (0, 0)),
            pl.BlockSpec((d_hid, d_out), lambda i: (0, 0)),
            pl.BlockSpec((1, d_out), lambda i: (0, 0)),
            pl.BlockSpec((1, d_out), lambda i: (0, 0)),
            pl.BlockSpec((1, d_out), lambda i: (0, 0)),
        ],
        out_specs=pl.BlockSpec((b_tile, seq, d_out), lambda i: (i, 0, 0)),
        out_shape=jax.ShapeDtypeStruct((batch, seq, d_out), jnp.float32),
    )(embs3, w1, b1, w2, b2, gamma, beta)


@jax.jit
def kernel(bboxes, coord_embed, W1, b1, W2, b2, gamma, beta):
    b, t, c = bboxes.shape
    embs = _sc_embed(bboxes.reshape(-1), coord_embed, b, t)
    embs3 = embs.reshape(b, _TPAD, c * _EMB)
    return _tc_mlp(embs3, W1, b1.reshape(1, -1), W2, b2.reshape(1, -1),
                   gamma.reshape(1, -1), beta.reshape(1, -1), b_tile=16)


# speedup vs baseline: 1.0290x; 1.0290x over previous
import functools

import jax
import jax.numpy as jnp
from jax import lax
from jax.experimental import pallas as pl
from jax.experimental.pallas import tpu as pltpu
from jax.experimental.pallas import tpu_sc as plsc

_BUCKETS = 128
_EMB = 32
_NW = 32
_CHUNK = 128
_LANES = 16


def _round_half_even_clip(y):
    k = y.astype(jnp.int32)
    r = y - k.astype(jnp.float32)
    half = jnp.float32(0.5)
    up = (r > half) | ((r == half) & ((k & 1) == 1))
    t = k + jnp.where(up, 1, 0)
    return jnp.minimum(jnp.maximum(t, 0), _BUCKETS - 1)


_TPAD = 56


def _sc_embed(flat_coords, coord_embed, batch, seq):
    b_per_w = batch // _NW
    b_per_c = 2
    rows_c = b_per_c * _TPAD
    n_chunks = b_per_w // b_per_c
    n_pairs = n_chunks // 2

    mesh = plsc.VectorSubcoreMesh(core_axis_name="c", subcore_axis_name="s")

    @functools.partial(
        pl.kernel,
        mesh=mesh,
        out_type=jax.ShapeDtypeStruct((batch * _TPAD, 4 * _EMB), jnp.float32),
        scratch_types=[
            pltpu.VMEM((b_per_w * seq * 4,), jnp.float32),
            pltpu.VMEM((_BUCKETS, _EMB), jnp.float32),
            pltpu.VMEM((7 * 4 * _LANES,), jnp.int32),
            pltpu.VMEM((rows_c, 4 * _EMB), jnp.float32),
            pltpu.VMEM((rows_c, 4 * _EMB), jnp.float32),
            pltpu.SemaphoreType.DMA,
        ],
        compiler_params=pltpu.CompilerParams(
            use_tc_tiling_on_sc=False, needs_layout_passes=False),
    )
    def k(coords_hbm, table_hbm, out_hbm, coords_v, table_v, idsb, ob0, ob1,
          sem):
        wid = lax.axis_index("s") * 2 + lax.axis_index("c")
        obufs = [ob0, ob1]
        iota = lax.iota(jnp.int32, _LANES)
        pltpu.sync_copy(table_hbm, table_v)
        pltpu.sync_copy(
            coords_hbm.at[pl.ds(wid * (b_per_w * seq * 4), b_per_w * seq * 4)],
            coords_v)

        def fill(obuf, j):
            cbase = j * (b_per_c * seq * 4)

            def ids_body(g, carry):
                rvec = iota + g * _LANES
                b_l = jnp.where(rvec >= _TPAD, 1, 0)
                t = jnp.minimum(rvec - b_l * _TPAD, seq - 1)
                cidx = cbase + b_l * (seq * 4) + t * 4
                for c in range(4):
                    xi = plsc.load_gather(coords_v, [cidx + c])
                    idsb[pl.ds(g * 4 * _LANES + c * _LANES, _LANES)] = (
                        _round_half_even_clip(xi * jnp.float32(_BUCKETS - 1)))
                return carry

            lax.fori_loop(0, rows_c // _LANES, ids_body, 0)

            def lk_body(g, carry):
                rows0 = g * _LANES
                base = g * 4 * _LANES
                for tl in range(_LANES):
                    for c in range(4):
                        spl = plsc.load_gather(
                            idsb, [jnp.full((_LANES,), 1, jnp.int32)
                                   * (base + c * _LANES + tl)])
                        for h in range(2):
                            v = plsc.load_gather(
                                table_v, [spl, iota + h * _LANES])
                            obuf[rows0 + tl,
                                 pl.ds(c * _EMB + h * _LANES, _LANES)] = v
                return carry

            lax.fori_loop(0, rows_c // _LANES, lk_body, 0)

        def pair_body(jj, carry):
            for b in range(2):
                j = jj * 2 + b

                @pl.when(jj > 0)
                def _wait():
                    pltpu.make_async_copy(
                        out_hbm.at[pl.ds(0, rows_c), :], obufs[b], sem).wait()

                fill(obufs[b], j)
                pltpu.async_copy(
                    obufs[b],
                    out_hbm.at[pl.ds(wid * (b_per_w * _TPAD) + j * rows_c,
                                     rows_c), :],
                    sem)
            return carry

        lax.fori_loop(0, n_pairs, pair_body, 0)
        for b in range(2):
            pltpu.make_async_copy(
                out_hbm.at[pl.ds(0, rows_c), :], obufs[b], sem).wait()

    return k(flat_coords, coord_embed)


def _tc_mlp(embs3, w1, b1, w2, b2, gamma, beta, b_tile):
    batch, tpad, d_in = embs3.shape
    seq = 50
    d_hid = w1.shape[1]
    d_out = w2.shape[1]

    def body(e_ref, w1_ref, b1_ref, w2_ref, b2_ref, g_ref, be_ref, o_ref):
        e = e_ref[...].reshape(b_tile * tpad, d_in)
        h = jnp.dot(e, w1_ref[...],
                    preferred_element_type=jnp.float32) + b1_ref[...]
        h = h * 0.5 * (1.0 + lax.erf(h * jnp.float32(0.7071067811865476)))
        y = jnp.dot(h, w2_ref[...],
                    preferred_element_type=jnp.float32) + b2_ref[...]
        mu = jnp.mean(y, axis=-1, keepdims=True)
        var = jnp.mean((y - mu) * (y - mu), axis=-1, keepdims=True)
        y = (y - mu) / jnp.sqrt(var + 1e-5) * g_ref[...] + be_ref[...]
        o_ref[...] = y.reshape(b_tile, tpad, d_out)[:, :seq, :]

    return pl.pallas_call(
        body,
        grid=(batch // b_tile,),
        in_specs=[
            pl.BlockSpec((b_tile, tpad, d_in), lambda i: (i, 0, 0)),
            pl.BlockSpec((d_in, d_hid), lambda i: (0, 0)),
            pl.BlockSpec((1, d_hid), lambda i: (0, 0)),
            pl.BlockSpec((d_hid, d_out), lambda i: (0, 0)),
            pl.BlockSpec((1, d_out), lambda i: (0, 0)),
            pl.BlockSpec((1, d_out), lambda i: (0, 0)),
            pl.BlockSpec((1, d_out), lambda i: (0, 0)),
        ],
        out_specs=pl.BlockSpec((b_tile, seq, d_out), lambda i: (i, 0, 0)),
        out_shape=jax.ShapeDtypeStruct((batch, seq, d_out), jnp.float32),
    )(embs3, w1, b1, w2, b2, gamma, beta)


@jax.jit
def kernel(bboxes, coord_embed, W1, b1, W2, b2, gamma, beta):
    b, t, c = bboxes.shape
    embs = _sc_embed(bboxes.reshape(-1), coord_embed, b, t)
    embs3 = embs.reshape(b, _TPAD, c * _EMB)
    return _tc_mlp(embs3, W1, b1.reshape(1, -1), W2, b2.reshape(1, -1),
                   gamma.reshape(1, -1), beta.reshape(1, -1), b_tile=16)


# speedup vs baseline: 1.1775x; 1.1443x over previous
import functools

import jax
import jax.numpy as jnp
from jax import lax
from jax.experimental import pallas as pl
from jax.experimental.pallas import tpu as pltpu
from jax.experimental.pallas import tpu_sc as plsc

_BUCKETS = 128
_EMB = 32
_NW = 32
_CHUNK = 128
_LANES = 16


def _round_half_even_clip(y):
    k = y.astype(jnp.int32)
    r = y - k.astype(jnp.float32)
    half = jnp.float32(0.5)
    up = (r > half) | ((r == half) & ((k & 1) == 1))
    t = k + jnp.where(up, 1, 0)
    return jnp.minimum(jnp.maximum(t, 0), _BUCKETS - 1)


_TPAD = 56


def _sc_embed(flat_coords, coord_embed, batch, seq):
    b_per_w = batch // _NW
    b_per_c = 2
    rows_c = b_per_c * _TPAD
    n_chunks = b_per_w // b_per_c
    n_pairs = n_chunks // 2

    mesh = plsc.VectorSubcoreMesh(core_axis_name="c", subcore_axis_name="s")

    @functools.partial(
        pl.kernel,
        mesh=mesh,
        out_type=jax.ShapeDtypeStruct((batch * _TPAD, 4 * _EMB), jnp.float32),
        scratch_types=[
            pltpu.VMEM((b_per_w * seq * 4,), jnp.float32),
            pltpu.VMEM((_BUCKETS, _EMB), jnp.float32),
            pltpu.VMEM((7 * 4 * _LANES,), jnp.int32),
            pltpu.VMEM((rows_c, 4 * _EMB), jnp.float32),
            pltpu.VMEM((rows_c, 4 * _EMB), jnp.float32),
            pltpu.SemaphoreType.DMA,
        ],
        compiler_params=pltpu.CompilerParams(
            use_tc_tiling_on_sc=False, needs_layout_passes=False),
    )
    def k(coords_hbm, table_hbm, out_hbm, coords_v, table_v, idsb, ob0, ob1,
          sem):
        wid = lax.axis_index("s") * 2 + lax.axis_index("c")
        obufs = [ob0, ob1]
        iota = lax.iota(jnp.int32, _LANES)
        pltpu.sync_copy(table_hbm, table_v)
        pltpu.sync_copy(
            coords_hbm.at[pl.ds(wid * (b_per_w * seq * 4), b_per_w * seq * 4)],
            coords_v)

        def fill(obuf, j):
            cbase = j * (b_per_c * seq * 4)

            def ids_body(g, carry):
                rvec = iota + g * _LANES
                b_l = jnp.where(rvec >= _TPAD, 1, 0)
                t = jnp.minimum(rvec - b_l * _TPAD, seq - 1)
                cidx = cbase + b_l * (seq * 4) + t * 4
                for c in range(4):
                    xi = plsc.load_gather(coords_v, [cidx + c])
                    idsb[pl.ds(g * 4 * _LANES + c * _LANES, _LANES)] = (
                        _round_half_even_clip(xi * jnp.float32(_BUCKETS - 1)))
                return carry

            lax.fori_loop(0, rows_c // _LANES, ids_body, 0)

            def lk_body(g, carry):
                rows0 = g * _LANES
                base = g * 4 * _LANES
                for tl in range(_LANES):
                    for c in range(4):
                        spl = plsc.load_gather(
                            idsb, [jnp.full((_LANES,), 1, jnp.int32)
                                   * (base + c * _LANES + tl)])
                        for h in range(2):
                            v = plsc.load_gather(
                                table_v, [spl, iota + h * _LANES])
                            obuf[rows0 + tl,
                                 pl.ds(c * _EMB + h * _LANES, _LANES)] = v
                return carry

            lax.fori_loop(0, rows_c // _LANES, lk_body, 0)

        def pair_body(jj, carry):
            for b in range(2):
                j = jj * 2 + b

                @pl.when(jj > 0)
                def _wait():
                    pltpu.make_async_copy(
                        out_hbm.at[pl.ds(0, rows_c), :], obufs[b], sem).wait()

                fill(obufs[b], j)
                pltpu.async_copy(
                    obufs[b],
                    out_hbm.at[pl.ds(wid * (b_per_w * _TPAD) + j * rows_c,
                                     rows_c), :],
                    sem)
            return carry

        lax.fori_loop(0, n_pairs, pair_body, 0)
        for b in range(2):
            pltpu.make_async_copy(
                out_hbm.at[pl.ds(0, rows_c), :], obufs[b], sem).wait()

    return k(flat_coords, coord_embed)


def _tc_mlp(embs3, w1, b1, w2, b2, gamma, beta, b_tile, out_batch, b_off,
            prev):
    batch, tpad, d_in = embs3.shape
    seq = 50
    d_hid = w1.shape[1]
    d_out = w2.shape[1]
    off = b_off // b_tile

    def body(e_ref, w1_ref, b1_ref, w2_ref, b2_ref, g_ref, be_ref, *rest):
        o_ref = rest[-1]
        e = e_ref[...].reshape(b_tile * tpad, d_in)
        h = jnp.dot(e, w1_ref[...],
                    preferred_element_type=jnp.float32) + b1_ref[...]
        h = h * 0.5 * (1.0 + lax.erf(h * jnp.float32(0.7071067811865476)))
        y = jnp.dot(h, w2_ref[...],
                    preferred_element_type=jnp.float32) + b2_ref[...]
        mu = jnp.mean(y, axis=-1, keepdims=True)
        var = jnp.mean((y - mu) * (y - mu), axis=-1, keepdims=True)
        y = (y - mu) / jnp.sqrt(var + 1e-5) * g_ref[...] + be_ref[...]
        o_ref[...] = y.reshape(b_tile, tpad, d_out)[:, :seq, :]

    in_specs = [
        pl.BlockSpec((b_tile, tpad, d_in), lambda i: (i, 0, 0)),
        pl.BlockSpec((d_in, d_hid), lambda i: (0, 0)),
        pl.BlockSpec((1, d_hid), lambda i: (0, 0)),
        pl.BlockSpec((d_hid, d_out), lambda i: (0, 0)),
        pl.BlockSpec((1, d_out), lambda i: (0, 0)),
        pl.BlockSpec((1, d_out), lambda i: (0, 0)),
        pl.BlockSpec((1, d_out), lambda i: (0, 0)),
    ]
    args = [embs3, w1, b1, w2, b2, gamma, beta]
    aliases = {}
    if prev is not None:
        in_specs.append(pl.BlockSpec(memory_space=pl.ANY))
        args.append(prev)
        aliases = {7: 0}

    return pl.pallas_call(
        body,
        grid=(batch // b_tile,),
        in_specs=in_specs,
        out_specs=pl.BlockSpec((b_tile, seq, d_out),
                               lambda i: (i + off, 0, 0)),
        out_shape=jax.ShapeDtypeStruct((out_batch, seq, d_out), jnp.float32),
        input_output_aliases=aliases,
    )(*args)


@jax.jit
def kernel(bboxes, coord_embed, W1, b1, W2, b2, gamma, beta):
    b, t, c = bboxes.shape
    flat = bboxes.reshape(-1)
    half = b // 2
    n_half = half * t * c
    w = (W1, b1.reshape(1, -1), W2, b2.reshape(1, -1),
         gamma.reshape(1, -1), beta.reshape(1, -1))
    embs_a = _sc_embed(flat[:n_half], coord_embed, half, t)
    embs_b = _sc_embed(flat[n_half:], coord_embed, half, t)
    y_a = _tc_mlp(embs_a.reshape(half, _TPAD, c * _EMB), *w, b_tile=16,
                  out_batch=b, b_off=0, prev=None)
    return _tc_mlp(embs_b.reshape(half, _TPAD, c * _EMB), *w, b_tile=16,
                   out_batch=b, b_off=half, prev=y_a)
